# Initial kernel scaffold; baseline (speedup 1.0000x reference)
import functools

import jax
import jax.numpy as jnp
from jax import lax
from jax.experimental import pallas as pl
from jax.experimental.pallas import tpu as pltpu
from jax.experimental.pallas import tpu_sc as plsc

N_ROWS = 32768
DIM = 256
N_CODES = 8192
CCOST = 0.25

BR = 1024
BK = 1024

NW = 32
BPW = N_ROWS // NW
CH = 128
NCH = BPW // CH


def _dist_argmin_body(x_ref, w_ref, idx_ref, dsum_ref, minv_ref, argm_ref):
    i = pl.program_id(0)
    j = pl.program_id(1)
    nk = pl.num_programs(1)
    x = x_ref[...]
    w = w_ref[...]
    dot = lax.dot_general(x, w, (((1,), (1,)), ((), ())),
                          preferred_element_type=jnp.float32)
    xsq = jnp.sum(x * x, axis=1)
    wsq = jnp.sum(w * w, axis=1)
    d = (xsq[:, None] + wsq[None, :]) - 2.0 * dot
    lmin = jnp.min(d, axis=1)
    iota = lax.broadcasted_iota(jnp.int32, (BR, BK), 1)
    larg = jnp.min(jnp.where(d == lmin[:, None], iota, N_CODES), axis=1)
    larg = larg + j * BK

    @pl.when(j == 0)
    def _():
        minv_ref[...] = lmin
        argm_ref[...] = larg

    @pl.when(j > 0)
    def _():
        better = lmin < minv_ref[...]
        minv_ref[...] = jnp.where(better, lmin, minv_ref[...])
        argm_ref[...] = jnp.where(better, larg, argm_ref[...])

    @pl.when(j == nk - 1)
    def _():
        idx_ref[...] = argm_ref[...]
        partial = jnp.sum(minv_ref[...])

        @pl.when(i == 0)
        def _():
            dsum_ref[0, 0] = partial

        @pl.when(i > 0)
        def _():
            dsum_ref[0, 0] = dsum_ref[0, 0] + partial


def _dist_argmin(x, w):
    ni, nk = N_ROWS // BR, N_CODES // BK
    return pl.pallas_call(
        _dist_argmin_body,
        grid=(ni, nk),
        in_specs=[
            pl.BlockSpec((BR, DIM), lambda i, j: (i, 0)),
            pl.BlockSpec((BK, DIM), lambda i, j: (j, 0)),
        ],
        out_specs=[
            pl.BlockSpec((BR,), lambda i, j: (i,)),
            pl.BlockSpec((1, 1), lambda i, j: (0, 0), memory_space=pltpu.SMEM),
        ],
        out_shape=[
            jax.ShapeDtypeStruct((N_ROWS,), jnp.int32),
            jax.ShapeDtypeStruct((1, 1), jnp.float32),
        ],
        scratch_shapes=[
            pltpu.VMEM((BR,), jnp.float32),
            pltpu.VMEM((BR,), jnp.int32),
        ],
        compiler_params=pltpu.CompilerParams(
            dimension_semantics=("arbitrary", "arbitrary")),
    )(x, w)


def _sc_body(table_hbm, idx_hbm, out_hbm, counts_hbm, idx_v, rows_v, bins_v,
             sem):
    wid = lax.axis_index("s") * 2 + lax.axis_index("c")

    def zbody(i, _):
        bins_v[pl.ds(i * 16, 16)] = jnp.zeros((16,), jnp.float32)
        return 0

    lax.fori_loop(0, N_CODES // 16, zbody, 0)

    base_w = wid * BPW
    for c in range(NCH):
        base = base_w + c * CH
        pltpu.sync_copy(idx_hbm.at[pl.ds(base, CH)], idx_v)
        pltpu.async_copy(table_hbm.at[idx_v], rows_v, sem).wait()
        pltpu.sync_copy(rows_v, out_hbm.at[pl.ds(base, CH)])

        def hbody(i, _):
            v = idx_v[i]
            bins_v[v] = bins_v[v] + 1.0
            return 0

        lax.fori_loop(0, CH, hbody, 0)

    pltpu.sync_copy(bins_v, counts_hbm.at[wid])


def _sc_gather_hist(table, idx):
    mesh = plsc.VectorSubcoreMesh(core_axis_name="c", subcore_axis_name="s")
    fn = functools.partial(
        pl.kernel,
        mesh=mesh,
        out_type=[
            jax.ShapeDtypeStruct((N_ROWS, DIM), jnp.float32),
            jax.ShapeDtypeStruct((NW, N_CODES), jnp.float32),
        ],
        scratch_types=[
            pltpu.VMEM((CH,), jnp.int32),
            pltpu.VMEM((CH, DIM), jnp.float32),
            pltpu.VMEM((N_CODES,), jnp.float32),
            pltpu.SemaphoreType.DMA,
        ],
    )(_sc_body)
    return fn(table, idx)


def _finalize_body(counts_ref, dsum_ref, loss_ref, perp_ref):
    counts = jnp.sum(counts_ref[...], axis=0)
    probs = counts * (1.0 / N_ROWS)
    ent = -jnp.sum(probs * jnp.log(probs + 1e-10))
    perp_ref[0, 0] = jnp.exp(ent)
    loss_ref[0, 0] = CCOST * dsum_ref[0, 0] / (N_ROWS * DIM)


def _finalize(counts, dsum):
    return pl.pallas_call(
        _finalize_body,
        in_specs=[
            pl.BlockSpec((NW, N_CODES), lambda: (0, 0)),
            pl.BlockSpec(memory_space=pltpu.SMEM),
        ],
        out_specs=[
            pl.BlockSpec(memory_space=pltpu.SMEM),
            pl.BlockSpec(memory_space=pltpu.SMEM),
        ],
        out_shape=[
            jax.ShapeDtypeStruct((1, 1), jnp.float32),
            jax.ShapeDtypeStruct((1, 1), jnp.float32),
        ],
    )(counts, dsum)


def kernel(inputs, embedding_weight):
    x = jnp.transpose(inputs, (0, 2, 3, 1)).reshape(N_ROWS, DIM)
    indices, dsum = _dist_argmin(x, embedding_weight)
    qflat, counts = _sc_gather_hist(embedding_weight, indices)
    loss, perp = _finalize(counts, dsum)
    q = qflat.reshape(8, 64, 64, DIM).transpose(0, 3, 1, 2)
    return (loss.reshape(()), q, perp.reshape(()), indices)


# speedup vs baseline: 1.0813x; 1.0813x over previous
import functools

import jax
import jax.numpy as jnp
from jax import lax
from jax.experimental import pallas as pl
from jax.experimental.pallas import tpu as pltpu
from jax.experimental.pallas import tpu_sc as plsc

N_ROWS = 32768
DIM = 256
N_CODES = 8192
CCOST = 0.25

BR = 256

NW = 32
BPW = N_ROWS // NW
CH = 128
NCH = BPW // CH


SEG_BOUNDS = (0, 2736, 5472, N_CODES)


def _dist_argmin_body(x_ref, wb_ref, wsq_ref, idx_ref, dsum_ref):
    i = pl.program_id(0)
    x = x_ref[...]
    dot = lax.dot_general(x.astype(jnp.bfloat16), wb_ref[...],
                          (((1,), (1,)), ((), ())),
                          preferred_element_type=jnp.float32)
    xsq = jnp.sum(x * x, axis=1)
    d = (xsq[:, None] + wsq_ref[...][None, :]) - 2.0 * dot
    iota = lax.broadcasted_iota(jnp.int32, (BR, N_CODES), 1)
    acc_v = None
    for s0, s1 in zip(SEG_BOUNDS[:-1], SEG_BOUNDS[1:]):
        mask = (iota >= s0) & (iota < s1)
        ds = jnp.where(mask, d, jnp.inf)
        m = jnp.min(ds, axis=1)
        a = jnp.min(jnp.where(ds == m[:, None], iota, N_CODES), axis=1)
        if acc_v is None:
            acc_v, acc_i, sel_v = m, a, m
        else:
            better = m < acc_v
            acc_i = jnp.where(better, a, acc_i)
            sel_v = jnp.where(better, m, sel_v)
            acc_v = jnp.where(better, m, acc_v)
        acc_v = acc_v.astype(jnp.bfloat16).astype(jnp.float32)
    idx_ref[...] = acc_i
    partial = jnp.sum(sel_v)

    @pl.when(i == 0)
    def _():
        dsum_ref[0, 0] = partial

    @pl.when(i > 0)
    def _():
        dsum_ref[0, 0] = dsum_ref[0, 0] + partial


def _wsq_body(w_ref, wsq_ref):
    w = w_ref[...]
    wsq_ref[...] = jnp.sum(w * w, axis=1)


def _dist_argmin(x, w):
    wsq = pl.pallas_call(
        _wsq_body,
        out_shape=jax.ShapeDtypeStruct((N_CODES,), jnp.float32),
    )(w)
    wb = w.astype(jnp.bfloat16)
    ni = N_ROWS // BR
    return pl.pallas_call(
        _dist_argmin_body,
        grid=(ni,),
        in_specs=[
            pl.BlockSpec((BR, DIM), lambda i: (i, 0)),
            pl.BlockSpec((N_CODES, DIM), lambda i: (0, 0)),
            pl.BlockSpec((N_CODES,), lambda i: (0,)),
        ],
        out_specs=[
            pl.BlockSpec((BR,), lambda i: (i,)),
            pl.BlockSpec((1, 1), lambda i: (0, 0), memory_space=pltpu.SMEM),
        ],
        out_shape=[
            jax.ShapeDtypeStruct((N_ROWS,), jnp.int32),
            jax.ShapeDtypeStruct((1, 1), jnp.float32),
        ],
        compiler_params=pltpu.CompilerParams(
            dimension_semantics=("arbitrary",)),
    )(x, wb, wsq)


def _sc_body(table_hbm, idx_hbm, out_hbm, counts_hbm, idx_v, rows_v, bins_v,
             sem):
    wid = lax.axis_index("s") * 2 + lax.axis_index("c")

    def zbody(i, _):
        bins_v[pl.ds(i * 16, 16)] = jnp.zeros((16,), jnp.float32)
        return 0

    lax.fori_loop(0, N_CODES // 16, zbody, 0)

    base_w = wid * BPW
    for c in range(NCH):
        base = base_w + c * CH
        pltpu.sync_copy(idx_hbm.at[pl.ds(base, CH)], idx_v)
        pltpu.async_copy(table_hbm.at[idx_v], rows_v, sem).wait()
        pltpu.sync_copy(rows_v, out_hbm.at[pl.ds(base, CH)])

        def hbody(i, _):
            v = idx_v[pl.ds(i * 16, 16)]
            plsc.addupdate_scatter(bins_v, [v], jnp.ones((16,), jnp.float32))
            return 0

        lax.fori_loop(0, CH // 16, hbody, 0)

    pltpu.sync_copy(bins_v, counts_hbm.at[wid])


def _sc_gather_hist(table, idx):
    mesh = plsc.VectorSubcoreMesh(core_axis_name="c", subcore_axis_name="s")
    fn = functools.partial(
        pl.kernel,
        mesh=mesh,
        out_type=[
            jax.ShapeDtypeStruct((N_ROWS, DIM), jnp.float32),
            jax.ShapeDtypeStruct((NW, N_CODES), jnp.float32),
        ],
        scratch_types=[
            pltpu.VMEM((CH,), jnp.int32),
            pltpu.VMEM((CH, DIM), jnp.float32),
            pltpu.VMEM((N_CODES,), jnp.float32),
            pltpu.SemaphoreType.DMA,
        ],
        compiler_params=pltpu.CompilerParams(needs_layout_passes=False),
    )(_sc_body)
    return fn(table, idx)


def _finalize_body(counts_ref, dsum_ref, loss_ref, perp_ref):
    counts = jnp.sum(counts_ref[...], axis=0)
    probs = counts * (1.0 / N_ROWS)
    ent = -jnp.sum(probs * jnp.log(probs + 1e-10))
    perp_ref[0, 0] = jnp.exp(ent)
    loss_ref[0, 0] = CCOST * dsum_ref[0, 0] / (N_ROWS * DIM)


def _finalize(counts, dsum):
    return pl.pallas_call(
        _finalize_body,
        in_specs=[
            pl.BlockSpec((NW, N_CODES), lambda: (0, 0)),
            pl.BlockSpec(memory_space=pltpu.SMEM),
        ],
        out_specs=[
            pl.BlockSpec(memory_space=pltpu.SMEM),
            pl.BlockSpec(memory_space=pltpu.SMEM),
        ],
        out_shape=[
            jax.ShapeDtypeStruct((1, 1), jnp.float32),
            jax.ShapeDtypeStruct((1, 1), jnp.float32),
        ],
    )(counts, dsum)


def kernel(inputs, embedding_weight):
    x = jnp.transpose(inputs, (0, 2, 3, 1)).reshape(N_ROWS, DIM)
    indices, dsum = _dist_argmin(x, embedding_weight)
    qflat, counts = _sc_gather_hist(embedding_weight, indices)
    loss, perp = _finalize(counts, dsum)
    q = qflat.reshape(8, 64, 64, DIM).transpose(0, 3, 1, 2)
    return (loss.reshape(()), q, perp.reshape(()), indices)


# speedup vs baseline: 1.4624x; 1.3525x over previous
import functools

import jax
import jax.numpy as jnp
from jax import lax
from jax.experimental import pallas as pl
from jax.experimental.pallas import tpu as pltpu
from jax.experimental.pallas import tpu_sc as plsc

N_ROWS = 32768
DIM = 256
N_CODES = 8192
CCOST = 0.25

BR = 256

NW = 32
BPW = N_ROWS // NW
CH = 128
NCH = BPW // CH


SEG_BOUNDS = (0, 2736, 5472, N_CODES)


def _dist_argmin_body(x_ref, wb_ref, wsq_ref, idx_ref, dsum_ref):
    i = pl.program_id(0)
    x = x_ref[...]
    dot = lax.dot_general(x.astype(jnp.bfloat16), wb_ref[...],
                          (((1,), (1,)), ((), ())),
                          preferred_element_type=jnp.float32)
    xsq = jnp.sum(x * x, axis=1)
    d = (xsq[:, None] + wsq_ref[...][None, :]) - 2.0 * dot
    acc_v = None
    for s0, s1 in zip(SEG_BOUNDS[:-1], SEG_BOUNDS[1:]):
        ds = d[:, s0:s1]
        m = jnp.min(ds, axis=1)
        iota = lax.broadcasted_iota(jnp.int32, (BR, s1 - s0), 1) + s0
        a = jnp.min(jnp.where(ds == m[:, None], iota, N_CODES), axis=1)
        if acc_v is None:
            acc_v, acc_i, sel_v = m, a, m
        else:
            better = m < acc_v
            acc_i = jnp.where(better, a, acc_i)
            sel_v = jnp.where(better, m, sel_v)
            acc_v = jnp.where(better, m, acc_v)
        acc_v = acc_v.astype(jnp.bfloat16).astype(jnp.float32)
    idx_ref[...] = acc_i
    partial = jnp.sum(sel_v)

    @pl.when(i == 0)
    def _():
        dsum_ref[0, 0] = partial

    @pl.when(i > 0)
    def _():
        dsum_ref[0, 0] = dsum_ref[0, 0] + partial


def _wsq_body(w_ref, wsq_ref):
    w = w_ref[...]
    wsq_ref[...] = jnp.sum(w * w, axis=1)


def _dist_argmin(x, w):
    wsq = pl.pallas_call(
        _wsq_body,
        out_shape=jax.ShapeDtypeStruct((N_CODES,), jnp.float32),
    )(w)
    wb = w.astype(jnp.bfloat16)
    ni = N_ROWS // BR
    return pl.pallas_call(
        _dist_argmin_body,
        grid=(ni,),
        in_specs=[
            pl.BlockSpec((BR, DIM), lambda i: (i, 0)),
            pl.BlockSpec((N_CODES, DIM), lambda i: (0, 0)),
            pl.BlockSpec((N_CODES,), lambda i: (0,)),
        ],
        out_specs=[
            pl.BlockSpec((BR,), lambda i: (i,)),
            pl.BlockSpec((1, 1), lambda i: (0, 0), memory_space=pltpu.SMEM),
        ],
        out_shape=[
            jax.ShapeDtypeStruct((N_ROWS,), jnp.int32),
            jax.ShapeDtypeStruct((1, 1), jnp.float32),
        ],
        compiler_params=pltpu.CompilerParams(
            dimension_semantics=("arbitrary",)),
    )(x, wb, wsq)


def _sc_body(table_hbm, idx_hbm, out_hbm, counts_hbm, idx_v, rows_v, bins_v,
             sem):
    wid = lax.axis_index("s") * 2 + lax.axis_index("c")

    def zbody(i, _):
        bins_v[pl.ds(i * 16, 16)] = jnp.zeros((16,), jnp.float32)
        return 0

    lax.fori_loop(0, N_CODES // 16, zbody, 0)

    base_w = wid * BPW
    for c in range(NCH):
        base = base_w + c * CH
        pltpu.sync_copy(idx_hbm.at[pl.ds(base, CH)], idx_v)
        pltpu.async_copy(table_hbm.at[idx_v], rows_v, sem).wait()
        pltpu.sync_copy(rows_v, out_hbm.at[pl.ds(base, CH)])

        def hbody(i, _):
            v = idx_v[pl.ds(i * 16, 16)]
            plsc.addupdate_scatter(bins_v, [v], jnp.ones((16,), jnp.float32))
            return 0

        lax.fori_loop(0, CH // 16, hbody, 0)

    pltpu.sync_copy(bins_v, counts_hbm.at[wid])


def _sc_gather_hist(table, idx):
    mesh = plsc.VectorSubcoreMesh(core_axis_name="c", subcore_axis_name="s")
    fn = functools.partial(
        pl.kernel,
        mesh=mesh,
        out_type=[
            jax.ShapeDtypeStruct((N_ROWS, DIM), jnp.float32),
            jax.ShapeDtypeStruct((NW, N_CODES), jnp.float32),
        ],
        scratch_types=[
            pltpu.VMEM((CH,), jnp.int32),
            pltpu.VMEM((CH, DIM), jnp.float32),
            pltpu.VMEM((N_CODES,), jnp.float32),
            pltpu.SemaphoreType.DMA,
        ],
        compiler_params=pltpu.CompilerParams(needs_layout_passes=False),
    )(_sc_body)
    return fn(table, idx)


def _finalize_body(counts_ref, dsum_ref, loss_ref, perp_ref):
    counts = jnp.sum(counts_ref[...], axis=0)
    probs = counts * (1.0 / N_ROWS)
    ent = -jnp.sum(probs * jnp.log(probs + 1e-10))
    perp_ref[0, 0] = jnp.exp(ent)
    loss_ref[0, 0] = CCOST * dsum_ref[0, 0] / (N_ROWS * DIM)


def _finalize(counts, dsum):
    return pl.pallas_call(
        _finalize_body,
        in_specs=[
            pl.BlockSpec((NW, N_CODES), lambda: (0, 0)),
            pl.BlockSpec(memory_space=pltpu.SMEM),
        ],
        out_specs=[
            pl.BlockSpec(memory_space=pltpu.SMEM),
            pl.BlockSpec(memory_space=pltpu.SMEM),
        ],
        out_shape=[
            jax.ShapeDtypeStruct((1, 1), jnp.float32),
            jax.ShapeDtypeStruct((1, 1), jnp.float32),
        ],
    )(counts, dsum)


def kernel(inputs, embedding_weight):
    x = jnp.transpose(inputs, (0, 2, 3, 1)).reshape(N_ROWS, DIM)
    indices, dsum = _dist_argmin(x, embedding_weight)
    qflat, counts = _sc_gather_hist(embedding_weight, indices)
    loss, perp = _finalize(counts, dsum)
    q = qflat.reshape(8, 64, 64, DIM).transpose(0, 3, 1, 2)
    return (loss.reshape(()), q, perp.reshape(()), indices)


# speedup vs baseline: 1.5273x; 1.0443x over previous
import functools

import jax
import jax.numpy as jnp
from jax import lax
from jax.experimental import pallas as pl
from jax.experimental.pallas import tpu as pltpu
from jax.experimental.pallas import tpu_sc as plsc

N_ROWS = 32768
DIM = 256
N_CODES = 8192
CCOST = 0.25

BR = 512

NW = 32
BPW = N_ROWS // NW
CH = 128
NCH = BPW // CH


SEG_BOUNDS = (0, 2736, 5472, N_CODES)


def _dist_argmin_body(x_ref, wb_ref, wsq_ref, idx_ref, dsum_ref):
    i = pl.program_id(0)
    x = x_ref[...]
    dot = lax.dot_general(x.astype(jnp.bfloat16), wb_ref[...],
                          (((1,), (1,)), ((), ())),
                          preferred_element_type=jnp.float32)
    xsq = jnp.sum(x * x, axis=1)
    d = (xsq[:, None] + wsq_ref[...][None, :]) - 2.0 * dot
    acc_v = None
    for s0, s1 in zip(SEG_BOUNDS[:-1], SEG_BOUNDS[1:]):
        ds = d[:, s0:s1]
        m = jnp.min(ds, axis=1)
        iota = lax.broadcasted_iota(jnp.int32, (BR, s1 - s0), 1) + s0
        a = jnp.min(jnp.where(ds == m[:, None], iota, N_CODES), axis=1)
        if acc_v is None:
            acc_v, acc_i, sel_v = m, a, m
        else:
            better = m < acc_v
            acc_i = jnp.where(better, a, acc_i)
            sel_v = jnp.where(better, m, sel_v)
            acc_v = jnp.where(better, m, acc_v)
        acc_v = acc_v.astype(jnp.bfloat16).astype(jnp.float32)
    idx_ref[...] = acc_i
    partial = jnp.sum(sel_v)

    @pl.when(i == 0)
    def _():
        dsum_ref[0, 0] = partial

    @pl.when(i > 0)
    def _():
        dsum_ref[0, 0] = dsum_ref[0, 0] + partial


def _wsq_body(w_ref, wsq_ref):
    w = w_ref[...]
    wsq_ref[...] = jnp.sum(w * w, axis=1)


def _dist_argmin(x, w):
    wsq = pl.pallas_call(
        _wsq_body,
        out_shape=jax.ShapeDtypeStruct((N_CODES,), jnp.float32),
    )(w)
    wb = w.astype(jnp.bfloat16)
    ni = N_ROWS // BR
    return pl.pallas_call(
        _dist_argmin_body,
        grid=(ni,),
        in_specs=[
            pl.BlockSpec((BR, DIM), lambda i: (i, 0)),
            pl.BlockSpec((N_CODES, DIM), lambda i: (0, 0)),
            pl.BlockSpec((N_CODES,), lambda i: (0,)),
        ],
        out_specs=[
            pl.BlockSpec((BR,), lambda i: (i,)),
            pl.BlockSpec((1, 1), lambda i: (0, 0), memory_space=pltpu.SMEM),
        ],
        out_shape=[
            jax.ShapeDtypeStruct((N_ROWS,), jnp.int32),
            jax.ShapeDtypeStruct((1, 1), jnp.float32),
        ],
        compiler_params=pltpu.CompilerParams(
            dimension_semantics=("arbitrary",)),
    )(x, wb, wsq)


def _sc_body(table_hbm, idx_hbm, out_hbm, counts_hbm, idx_v, rows_v, bins_v,
             sem):
    wid = lax.axis_index("s") * 2 + lax.axis_index("c")

    def zbody(i, _):
        bins_v[pl.ds(i * 16, 16)] = jnp.zeros((16,), jnp.float32)
        return 0

    lax.fori_loop(0, N_CODES // 16, zbody, 0)

    base_w = wid * BPW
    for c in range(NCH):
        base = base_w + c * CH
        pltpu.sync_copy(idx_hbm.at[pl.ds(base, CH)], idx_v)
        pltpu.async_copy(table_hbm.at[idx_v], rows_v, sem).wait()
        pltpu.sync_copy(rows_v, out_hbm.at[pl.ds(base, CH)])

        def hbody(i, _):
            v = idx_v[pl.ds(i * 16, 16)]
            plsc.addupdate_scatter(bins_v, [v], jnp.ones((16,), jnp.float32))
            return 0

        lax.fori_loop(0, CH // 16, hbody, 0)

    pltpu.sync_copy(bins_v, counts_hbm.at[wid])


def _sc_gather_hist(table, idx):
    mesh = plsc.VectorSubcoreMesh(core_axis_name="c", subcore_axis_name="s")
    fn = functools.partial(
        pl.kernel,
        mesh=mesh,
        out_type=[
            jax.ShapeDtypeStruct((N_ROWS, DIM), jnp.float32),
            jax.ShapeDtypeStruct((NW, N_CODES), jnp.float32),
        ],
        scratch_types=[
            pltpu.VMEM((CH,), jnp.int32),
            pltpu.VMEM((CH, DIM), jnp.float32),
            pltpu.VMEM((N_CODES,), jnp.float32),
            pltpu.SemaphoreType.DMA,
        ],
        compiler_params=pltpu.CompilerParams(needs_layout_passes=False),
    )(_sc_body)
    return fn(table, idx)


def _finalize_body(counts_ref, dsum_ref, loss_ref, perp_ref):
    counts = jnp.sum(counts_ref[...], axis=0)
    probs = counts * (1.0 / N_ROWS)
    ent = -jnp.sum(probs * jnp.log(probs + 1e-10))
    perp_ref[0, 0] = jnp.exp(ent)
    loss_ref[0, 0] = CCOST * dsum_ref[0, 0] / (N_ROWS * DIM)


def _finalize(counts, dsum):
    return pl.pallas_call(
        _finalize_body,
        in_specs=[
            pl.BlockSpec((NW, N_CODES), lambda: (0, 0)),
            pl.BlockSpec(memory_space=pltpu.SMEM),
        ],
        out_specs=[
            pl.BlockSpec(memory_space=pltpu.SMEM),
            pl.BlockSpec(memory_space=pltpu.SMEM),
        ],
        out_shape=[
            jax.ShapeDtypeStruct((1, 1), jnp.float32),
            jax.ShapeDtypeStruct((1, 1), jnp.float32),
        ],
    )(counts, dsum)


def kernel(inputs, embedding_weight):
    x = jnp.transpose(inputs, (0, 2, 3, 1)).reshape(N_ROWS, DIM)
    indices, dsum = _dist_argmin(x, embedding_weight)
    qflat, counts = _sc_gather_hist(embedding_weight, indices)
    loss, perp = _finalize(counts, dsum)
    q = qflat.reshape(8, 64, 64, DIM).transpose(0, 3, 1, 2)
    return (loss.reshape(()), q, perp.reshape(()), indices)
